# Initial kernel scaffold; baseline (speedup 1.0000x reference)
import functools

import jax
import jax.numpy as jnp
from jax import lax
from jax.experimental import pallas as pl
from jax.experimental.pallas import tpu as pltpu
from jax.experimental.pallas import tpu_sc as plsc

N_NODES = 100000
N_EDGES = 3200000
NUM_REL = 16
D_IN = 7
D_HID = 16
D_OUT = 2
D_OUT_PAD = 8

NC, NS = 2, 16
NW = NC * NS
EB = 128
EPAD = 3276800
RTOT = EPAD // EB
ROWS_PER_TILE = RTOT // NW
NPAD = N_NODES + 16
RPT_OUT = NPAD // NS

BN = 2000
GRID_N = N_NODES // BN

ZROWS = 1024
Z1DL = 8192


def _idx_body(src_ref, et_ref, idx_ref):
    idx_ref[...] = et_ref[...] * N_NODES + src_ref[...]


def _tc_idx(srcp, etp):
    blk = pl.BlockSpec((512, EB), lambda i: (i, 0))
    return pl.pallas_call(
        _idx_body,
        grid=(RTOT // 512,),
        in_specs=[blk, blk],
        out_specs=blk,
        out_shape=jax.ShapeDtypeStruct((RTOT, EB), jnp.int32),
    )(srcp, etp)


def _prep_body(x_ref, w_ref, y_ref):
    xb = x_ref[...]
    for r in range(NUM_REL):
        y_ref[r] = jnp.dot(xb, w_ref[r], preferred_element_type=jnp.float32)


def _tc_prep(x, w1):
    return pl.pallas_call(
        _prep_body,
        grid=(GRID_N,),
        in_specs=[
            pl.BlockSpec((BN, D_IN), lambda i: (i, 0)),
            pl.BlockSpec((NUM_REL, D_IN, D_HID), lambda i: (0, 0, 0)),
        ],
        out_specs=pl.BlockSpec((NUM_REL, BN, D_HID), lambda i: (0, i, 0)),
        out_shape=jax.ShapeDtypeStruct((NUM_REL, N_NODES, D_HID),
                                       jnp.float32),
    )(x, w1)


def _make_sc_pass(width, with_cnt, kb):
    mesh = plsc.VectorSubcoreMesh(core_axis_name="c", subcore_axis_name="s",
                                  num_cores=NC, num_subcores=NS)
    outer = ROWS_PER_TILE // kb

    scratch = [
        pltpu.VMEM((kb, EB), jnp.int32),
        pltpu.VMEM((kb, EB), jnp.int32),
        pltpu.VMEM((kb * EB, width), jnp.float32),
        pltpu.VMEM((ZROWS, width), jnp.float32),
        pltpu.VMEM_SHARED((NPAD, width), jnp.float32),
        pltpu.SemaphoreType.DMA,
    ]
    out_type = [jax.ShapeDtypeStruct((NC, NPAD, width), jnp.float32)]
    if with_cnt:
        scratch += [
            pltpu.VMEM((EB,), jnp.float32),
            pltpu.VMEM((Z1DL,), jnp.float32),
            pltpu.VMEM_SHARED((NPAD,), jnp.float32),
        ]
        out_type.append(jax.ShapeDtypeStruct((NC, NPAD), jnp.float32))

    @functools.partial(pl.kernel, out_type=out_type, mesh=mesh,
                       scratch_types=scratch)
    def sc_pass(idx_hbm, dst_hbm, tab_hbm, zrow_hbm, z1d_hbm, *refs):
        if with_cnt:
            (sums_hbm, cnts_hbm, idx_v, dst_v, rows_v, zv, acc_sh, sem,
             ones_v, z1_v, cnt_sh) = refs
        else:
            (sums_hbm, idx_v, dst_v, rows_v, zv, acc_sh, sem) = refs
        c = lax.axis_index("c")
        s = lax.axis_index("s")
        wid = c * NS + s

        pltpu.sync_copy(zrow_hbm, zv)
        zbase = s * RPT_OUT
        nfull = RPT_OUT // ZROWS
        for k in range(nfull):
            pltpu.sync_copy(zv, acc_sh.at[pl.ds(zbase + k * ZROWS, ZROWS)])
        rem = RPT_OUT - nfull * ZROWS
        pltpu.sync_copy(zv.at[pl.ds(0, rem)],
                        acc_sh.at[pl.ds(zbase + nfull * ZROWS, rem)])
        if with_cnt:
            pltpu.sync_copy(z1d_hbm, z1_v)

            @pl.when(s == 0)
            def _zero_cnt():
                nf1 = NPAD // Z1DL
                for k in range(nf1):
                    pltpu.sync_copy(z1_v, cnt_sh.at[pl.ds(k * Z1DL, Z1DL)])
                r1 = NPAD - nf1 * Z1DL
                pltpu.sync_copy(z1_v.at[pl.ds(0, r1)],
                                cnt_sh.at[pl.ds(nf1 * Z1DL, r1)])

            def _init_ones(i, carry):
                ones_v[pl.ds(i * 16, 16)] = jnp.full((16,), 1.0, jnp.float32)
                return carry
            lax.fori_loop(0, EB // 16, _init_ones, 0)

        plsc.subcore_barrier()

        def _step(jo, carry):
            r0 = wid * ROWS_PER_TILE + jo * kb
            pltpu.sync_copy(idx_hbm.at[pl.ds(r0, kb)], idx_v)
            pltpu.sync_copy(dst_hbm.at[pl.ds(r0, kb)], dst_v)
            cps = [pltpu.async_copy(tab_hbm.at[idx_v.at[j]],
                                    rows_v.at[pl.ds(j * EB, EB)], sem)
                   for j in range(kb)]
            for cp in cps:
                cp.wait()
            for j in range(kb):
                pltpu.sync_copy(rows_v.at[pl.ds(j * EB, EB)],
                                acc_sh.at[dst_v.at[j]], add=True)
                if with_cnt:
                    pltpu.sync_copy(ones_v, cnt_sh.at[dst_v.at[j]],
                                    add=True)
            return carry
        lax.fori_loop(0, outer, _step, 0)

        plsc.subcore_barrier()

        ob = s * RPT_OUT
        pltpu.sync_copy(acc_sh.at[pl.ds(ob, RPT_OUT)],
                        sums_hbm.at[c, pl.ds(ob, RPT_OUT)])
        if with_cnt:
            @pl.when(s == 0)
            def _cnt_out():
                pltpu.sync_copy(cnt_sh, cnts_hbm.at[c])

    return sc_pass


_sc_pass1 = _make_sc_pass(D_HID, True, 4)
_sc_pass2 = _make_sc_pass(D_OUT_PAD, False, 8)


def _mid_body(sums_ref, cnts_ref, x_ref, root1_ref, b1_ref, w2_ref,
              root2_ref, b2_ref, y2_ref, xr2_ref):
    agg = sums_ref[0] + sums_ref[1]
    cnt = cnts_ref[0, :] + cnts_ref[1, :]
    inv = 1.0 / jnp.maximum(cnt, 1.0)
    h = agg * inv[:, None] + jnp.dot(
        x_ref[...], root1_ref[...], preferred_element_type=jnp.float32)
    h = jnp.maximum(h + b1_ref[...], 0.0)
    for r in range(NUM_REL):
        y2_ref[r] = jnp.dot(h, w2_ref[r], preferred_element_type=jnp.float32)
    xr2_ref[...] = jnp.dot(h, root2_ref[...],
                           preferred_element_type=jnp.float32) + b2_ref[...]


def _tc_mid(sums1, cnts, x, root1, b1r, w2p, root2p, b2r):
    return pl.pallas_call(
        _mid_body,
        grid=(GRID_N,),
        in_specs=[
            pl.BlockSpec((NC, BN, D_HID), lambda i: (0, i, 0)),
            pl.BlockSpec((NC, BN), lambda i: (0, i)),
            pl.BlockSpec((BN, D_IN), lambda i: (i, 0)),
            pl.BlockSpec((D_IN, D_HID), lambda i: (0, 0)),
            pl.BlockSpec((1, D_HID), lambda i: (0, 0)),
            pl.BlockSpec((NUM_REL, D_HID, D_OUT_PAD), lambda i: (0, 0, 0)),
            pl.BlockSpec((D_HID, D_OUT_PAD), lambda i: (0, 0)),
            pl.BlockSpec((1, D_OUT_PAD), lambda i: (0, 0)),
        ],
        out_specs=[
            pl.BlockSpec((NUM_REL, BN, D_OUT_PAD), lambda i: (0, i, 0)),
            pl.BlockSpec((BN, D_OUT_PAD), lambda i: (i, 0)),
        ],
        out_shape=[
            jax.ShapeDtypeStruct((NUM_REL, N_NODES, D_OUT_PAD), jnp.float32),
            jax.ShapeDtypeStruct((N_NODES, D_OUT_PAD), jnp.float32),
        ],
    )(sums1, cnts, x, root1, b1r, w2p, root2p, b2r)


def _final_body(sums_ref, cnts_ref, xr2_ref, out_ref):
    agg = sums_ref[0] + sums_ref[1]
    cnt = cnts_ref[0, :] + cnts_ref[1, :]
    inv = 1.0 / jnp.maximum(cnt, 1.0)
    z = agg * inv[:, None] + xr2_ref[...]
    lane = lax.broadcasted_iota(jnp.int32, z.shape, 1)
    zm = jnp.where(lane < D_OUT, z, -jnp.inf)
    m = jnp.max(zm, axis=1, keepdims=True)
    lse = m + jnp.log(jnp.sum(jnp.where(lane < D_OUT, jnp.exp(z - m), 0.0),
                              axis=1, keepdims=True))
    out_ref[...] = (z - lse)[:, :D_OUT]


def _tc_final(sums2, cnts, xr2):
    return pl.pallas_call(
        _final_body,
        grid=(GRID_N,),
        in_specs=[
            pl.BlockSpec((NC, BN, D_OUT_PAD), lambda i: (0, i, 0)),
            pl.BlockSpec((NC, BN), lambda i: (0, i)),
            pl.BlockSpec((BN, D_OUT_PAD), lambda i: (i, 0)),
        ],
        out_specs=pl.BlockSpec((BN, D_OUT), lambda i: (i, 0)),
        out_shape=jax.ShapeDtypeStruct((N_NODES, D_OUT), jnp.float32),
    )(sums2, cnts, xr2)


def kernel(x, edge_index, edge_type, W1, root1, b1, W2, root2, b2):
    pad = EPAD - N_EDGES
    srcp = jnp.pad(edge_index[0], (0, pad)).reshape(RTOT, EB)
    etp = jnp.pad(edge_type, (0, pad)).reshape(RTOT, EB)
    dstp = jnp.pad(edge_index[1], (0, pad),
                   constant_values=N_NODES).reshape(RTOT, EB)

    idxp = _tc_idx(srcp, etp)
    y1 = _tc_prep(x, W1)

    zrow16 = jnp.zeros((ZROWS, D_HID), jnp.float32)
    z1d = jnp.zeros((Z1DL,), jnp.float32)
    sums1, cnts = _sc_pass1(idxp, dstp, y1.reshape(NUM_REL * N_NODES, D_HID),
                            zrow16, z1d)

    w2p = jnp.pad(W2, ((0, 0), (0, 0), (0, D_OUT_PAD - D_OUT)))
    root2p = jnp.pad(root2, ((0, 0), (0, D_OUT_PAD - D_OUT)))
    b2r = jnp.pad(b2, (0, D_OUT_PAD - D_OUT)).reshape(1, D_OUT_PAD)
    y2, xr2 = _tc_mid(sums1, cnts, x, root1, b1.reshape(1, D_HID),
                      w2p, root2p, b2r)

    zrow8 = jnp.zeros((ZROWS, D_OUT_PAD), jnp.float32)
    (sums2,) = _sc_pass2(idxp, dstp,
                         y2.reshape(NUM_REL * N_NODES, D_OUT_PAD), zrow8,
                         z1d)
    return _tc_final(sums2, cnts, xr2)


# speedup vs baseline: 34.3610x; 34.3610x over previous
import functools

import jax
import jax.numpy as jnp
from jax import lax
from jax.experimental import pallas as pl
from jax.experimental.pallas import tpu as pltpu
from jax.experimental.pallas import tpu_sc as plsc

N_NODES = 100000
N_EDGES = 3200000
NUM_REL = 16
D_IN = 7
D_HID = 16
D_OUT = 2
D_OUT_PAD = 8

NC, NS = 2, 16
NW = NC * NS
EB = 128
EPAD = 3276800
RTOT = EPAD // EB
ROWS_PER_TILE = RTOT // NW
NPAD = N_NODES + 96
RPT_OUT = NPAD // NS

BN = 2000
GRID_N = N_NODES // BN

ZROWS = 1024
Z1DL = 8192


def _idx_body(src_ref, et_ref, idx_ref):
    idx_ref[...] = et_ref[...] * N_NODES + src_ref[...]


def _tc_idx(srcp, etp):
    blk = pl.BlockSpec((512, EB), lambda i: (i, 0))
    return pl.pallas_call(
        _idx_body,
        grid=(RTOT // 512,),
        in_specs=[blk, blk],
        out_specs=blk,
        out_shape=jax.ShapeDtypeStruct((RTOT, EB), jnp.int32),
    )(srcp, etp)


def _prep_body(x_ref, w_ref, y_ref):
    xb = x_ref[...]
    for r in range(NUM_REL):
        y_ref[r] = jnp.dot(xb, w_ref[r], preferred_element_type=jnp.float32)


def _tc_prep(x, w1):
    return pl.pallas_call(
        _prep_body,
        grid=(GRID_N,),
        in_specs=[
            pl.BlockSpec((BN, D_IN), lambda i: (i, 0)),
            pl.BlockSpec((NUM_REL, D_IN, D_HID), lambda i: (0, 0, 0)),
        ],
        out_specs=pl.BlockSpec((NUM_REL, BN, D_HID), lambda i: (0, i, 0)),
        out_shape=jax.ShapeDtypeStruct((NUM_REL, N_NODES, D_HID),
                                       jnp.float32),
    )(x, w1)


def _make_sc_pass(width, kb):
    mesh = plsc.VectorSubcoreMesh(core_axis_name="c", subcore_axis_name="s",
                                  num_cores=NC, num_subcores=NS)
    outer = ROWS_PER_TILE // kb

    scratch = [
        pltpu.VMEM((kb, EB), jnp.int32),
        pltpu.VMEM((kb, EB), jnp.int32),
        pltpu.VMEM((kb * EB, width), jnp.float32),
        pltpu.VMEM((ZROWS, width), jnp.float32),
        pltpu.VMEM_SHARED((NPAD, width), jnp.float32),
        pltpu.SemaphoreType.DMA,
    ]
    out_type = [jax.ShapeDtypeStruct((NC, NPAD, width), jnp.float32)]

    @functools.partial(
        pl.kernel, out_type=out_type, mesh=mesh, scratch_types=scratch,
        compiler_params=pltpu.CompilerParams(use_tc_tiling_on_sc=False))
    def sc_pass(idx_hbm, dst_hbm, tab_hbm, zrow_hbm, *refs):
        (sums_hbm, idx_v, dst_v, rows_v, zv, acc_sh, sem) = refs
        c = lax.axis_index("c")
        s = lax.axis_index("s")
        wid = c * NS + s

        pltpu.sync_copy(zrow_hbm, zv)
        zbase = s * RPT_OUT
        nfull = RPT_OUT // ZROWS
        for k in range(nfull):
            pltpu.sync_copy(zv, acc_sh.at[pl.ds(zbase + k * ZROWS, ZROWS)])
        rem = RPT_OUT - nfull * ZROWS
        pltpu.sync_copy(zv.at[pl.ds(0, rem)],
                        acc_sh.at[pl.ds(zbase + nfull * ZROWS, rem)])

        plsc.subcore_barrier()

        def _step(jo, carry):
            r0 = wid * ROWS_PER_TILE + jo * kb
            pltpu.sync_copy(idx_hbm.at[pl.ds(r0, kb)], idx_v)
            pltpu.sync_copy(dst_hbm.at[pl.ds(r0, kb)], dst_v)
            cps = [pltpu.async_copy(tab_hbm.at[idx_v.at[j]],
                                    rows_v.at[pl.ds(j * EB, EB)], sem)
                   for j in range(kb)]
            for cp in cps:
                cp.wait()
            for j in range(kb):
                pltpu.sync_copy(rows_v.at[pl.ds(j * EB, EB)],
                                acc_sh.at[dst_v.at[j]], add=True)
            return carry
        lax.fori_loop(0, outer, _step, 0)

        plsc.subcore_barrier()

        ob = s * RPT_OUT
        pltpu.sync_copy(acc_sh.at[pl.ds(ob, RPT_OUT)],
                        sums_hbm.at[c, pl.ds(ob, RPT_OUT)])

    return sc_pass


_sc_pass1 = _make_sc_pass(D_HID, 4)
_sc_pass2 = _make_sc_pass(D_OUT_PAD, 8)


def _make_sc_cnt(kb):
    mesh = plsc.VectorSubcoreMesh(core_axis_name="c", subcore_axis_name="s",
                                  num_cores=NC, num_subcores=NS)
    outer = ROWS_PER_TILE // kb
    scratch = [
        pltpu.VMEM((kb, EB), jnp.int32),
        pltpu.VMEM((EB,), jnp.float32),
        pltpu.VMEM((Z1DL,), jnp.float32),
        pltpu.VMEM_SHARED((NPAD,), jnp.float32),
    ]
    out_type = [jax.ShapeDtypeStruct((NPAD,), jnp.float32),
                jax.ShapeDtypeStruct((NPAD,), jnp.float32)]

    @functools.partial(
        pl.kernel, out_type=out_type, mesh=mesh, scratch_types=scratch,
        compiler_params=pltpu.CompilerParams(use_tc_tiling_on_sc=False))
    def sc_cnt(dst_hbm, z1d_hbm, cnts0_hbm, cnts1_hbm, dst_v, ones_v, z1_v,
               cnt_sh):
        c = lax.axis_index("c")
        s = lax.axis_index("s")
        wid = c * NS + s

        pltpu.sync_copy(z1d_hbm, z1_v)

        @pl.when(s == 0)
        def _zero_cnt():
            nf1 = NPAD // Z1DL
            for k in range(nf1):
                pltpu.sync_copy(z1_v, cnt_sh.at[pl.ds(k * Z1DL, Z1DL)])
            r1 = NPAD - nf1 * Z1DL
            pltpu.sync_copy(z1_v.at[pl.ds(0, r1)],
                            cnt_sh.at[pl.ds(nf1 * Z1DL, r1)])

        def _init_ones(i, carry):
            ones_v[pl.ds(i * 16, 16)] = jnp.full((16,), 1.0, jnp.float32)
            return carry
        lax.fori_loop(0, EB // 16, _init_ones, 0)

        plsc.subcore_barrier()

        def _step(jo, carry):
            r0 = wid * ROWS_PER_TILE + jo * kb
            pltpu.sync_copy(dst_hbm.at[pl.ds(r0, kb)], dst_v)
            for j in range(kb):
                pltpu.sync_copy(ones_v, cnt_sh.at[dst_v.at[j]], add=True)
            return carry
        lax.fori_loop(0, outer, _step, 0)

        plsc.subcore_barrier()

        @pl.when((s == 0) & (c == 0))
        def _cnt_out0():
            pltpu.sync_copy(cnt_sh, cnts0_hbm)

        @pl.when((s == 0) & (c == 1))
        def _cnt_out1():
            pltpu.sync_copy(cnt_sh, cnts1_hbm)

    return sc_cnt


_sc_cnt = _make_sc_cnt(8)


def _mid_body(sums_ref, cnts_ref, x_ref, root1_ref, b1_ref, w2_ref,
              root2_ref, b2_ref, y2_ref, xr2_ref):
    agg = sums_ref[0] + sums_ref[1]
    cnt = cnts_ref[0, :, 0] + cnts_ref[1, :, 0]
    inv = 1.0 / jnp.maximum(cnt, 1.0)
    h = agg * inv[:, None] + jnp.dot(
        x_ref[...], root1_ref[...], preferred_element_type=jnp.float32)
    h = jnp.maximum(h + b1_ref[...], 0.0)
    for r in range(NUM_REL):
        y2_ref[r] = jnp.dot(h, w2_ref[r], preferred_element_type=jnp.float32)
    xr2_ref[...] = jnp.dot(h, root2_ref[...],
                           preferred_element_type=jnp.float32) + b2_ref[...]


def _tc_mid(sums1, cnts, x, root1, b1r, w2p, root2p, b2r):
    return pl.pallas_call(
        _mid_body,
        grid=(GRID_N,),
        in_specs=[
            pl.BlockSpec((NC, BN, D_HID), lambda i: (0, i, 0)),
            pl.BlockSpec((NC, BN, 1), lambda i: (0, i, 0)),
            pl.BlockSpec((BN, D_IN), lambda i: (i, 0)),
            pl.BlockSpec((D_IN, D_HID), lambda i: (0, 0)),
            pl.BlockSpec((1, D_HID), lambda i: (0, 0)),
            pl.BlockSpec((NUM_REL, D_HID, D_OUT_PAD), lambda i: (0, 0, 0)),
            pl.BlockSpec((D_HID, D_OUT_PAD), lambda i: (0, 0)),
            pl.BlockSpec((1, D_OUT_PAD), lambda i: (0, 0)),
        ],
        out_specs=[
            pl.BlockSpec((NUM_REL, BN, D_OUT_PAD), lambda i: (0, i, 0)),
            pl.BlockSpec((BN, D_OUT_PAD), lambda i: (i, 0)),
        ],
        out_shape=[
            jax.ShapeDtypeStruct((NUM_REL, N_NODES, D_OUT_PAD), jnp.float32),
            jax.ShapeDtypeStruct((N_NODES, D_OUT_PAD), jnp.float32),
        ],
    )(sums1, cnts, x, root1, b1r, w2p, root2p, b2r)


def _final_body(sums_ref, cnts_ref, xr2_ref, out_ref):
    agg = sums_ref[0] + sums_ref[1]
    cnt = cnts_ref[0, :, 0] + cnts_ref[1, :, 0]
    inv = 1.0 / jnp.maximum(cnt, 1.0)
    z = agg * inv[:, None] + xr2_ref[...]
    lane = lax.broadcasted_iota(jnp.int32, z.shape, 1)
    zm = jnp.where(lane < D_OUT, z, -jnp.inf)
    m = jnp.max(zm, axis=1, keepdims=True)
    lse = m + jnp.log(jnp.sum(jnp.where(lane < D_OUT, jnp.exp(z - m), 0.0),
                              axis=1, keepdims=True))
    out_ref[...] = (z - lse)[:, :D_OUT]


def _tc_final(sums2, cnts, xr2):
    return pl.pallas_call(
        _final_body,
        grid=(GRID_N,),
        in_specs=[
            pl.BlockSpec((NC, BN, D_OUT_PAD), lambda i: (0, i, 0)),
            pl.BlockSpec((NC, BN, 1), lambda i: (0, i, 0)),
            pl.BlockSpec((BN, D_OUT_PAD), lambda i: (i, 0)),
        ],
        out_specs=pl.BlockSpec((BN, D_OUT), lambda i: (i, 0)),
        out_shape=jax.ShapeDtypeStruct((N_NODES, D_OUT), jnp.float32),
    )(sums2, cnts, xr2)


def kernel(x, edge_index, edge_type, W1, root1, b1, W2, root2, b2):
    pad = EPAD - N_EDGES
    srcp = jnp.pad(edge_index[0], (0, pad)).reshape(RTOT, EB)
    etp = jnp.pad(edge_type, (0, pad)).reshape(RTOT, EB)
    dstp = jnp.pad(edge_index[1], (0, pad),
                   constant_values=N_NODES).reshape(RTOT, EB)

    idxp = _tc_idx(srcp, etp)
    y1 = _tc_prep(x, W1)

    zrow16 = jnp.zeros((ZROWS, D_HID), jnp.float32)
    z1d = jnp.zeros((Z1DL,), jnp.float32)
    cnts0, cnts1 = _sc_cnt(dstp, z1d)
    cnts3 = jnp.stack([cnts0, cnts1]).reshape(NC, NPAD, 1)
    (sums1,) = _sc_pass1(idxp, dstp,
                         y1.reshape(NUM_REL * N_NODES, D_HID), zrow16)

    w2p = jnp.pad(W2, ((0, 0), (0, 0), (0, D_OUT_PAD - D_OUT)))
    root2p = jnp.pad(root2, ((0, 0), (0, D_OUT_PAD - D_OUT)))
    b2r = jnp.pad(b2, (0, D_OUT_PAD - D_OUT)).reshape(1, D_OUT_PAD)
    y2, xr2 = _tc_mid(sums1, cnts3, x, root1, b1.reshape(1, D_HID),
                      w2p, root2p, b2r)

    zrow8 = jnp.zeros((ZROWS, D_OUT_PAD), jnp.float32)
    (sums2,) = _sc_pass2(idxp, dstp,
                         y2.reshape(NUM_REL * N_NODES, D_OUT_PAD), zrow8)
    return _tc_final(sums2, cnts3, xr2)
